# Initial kernel scaffold; baseline (speedup 1.0000x reference)
import jax
import jax.numpy as jnp
from jax.experimental import pallas as pl


def kernel(x, edge_index, edge_weight, W):
    raise NotImplementedError("write your pallas kernel here")


# speedup vs baseline: 4.4726x; 4.4726x over previous
import functools

import jax
import jax.numpy as jnp
from jax import lax
from jax.experimental import pallas as pl
from jax.experimental.pallas import tpu as pltpu
from jax.experimental.pallas import tpu_sc as plsc

N = 10000
E = 320000
D = 128
NC = 2
NS = 16
NW = NC * NS
EPT = E // NW
CHUNK = 80
NCHUNK = EPT // CHUNK
NPAD = 10240
RPT = NPAD // NS
ZROWS = 128
LANES = D // 16


def _matmul_body(x_ref, w_ref, o_ref):
    o_ref[...] = jnp.dot(x_ref[...], w_ref[...],
                         preferred_element_type=jnp.float32)


def _combine_body(p_ref, o_ref):
    o_ref[...] = jnp.maximum(p_ref[0] + p_ref[1], 0.0)


def _sc_body(presup_hbm, src_hbm, dst_hbm, w_hbm, out_hbm,
             src_v, dst_v, w_v, rows_v, zbuf_v, acc_sh, sem):
    c = lax.axis_index("c")
    s = lax.axis_index("s")
    wid = c * NS + s

    def _zero_zbuf(i, _):
        for j in range(LANES):
            zbuf_v[i, pl.ds(j * 16, 16)] = jnp.zeros((16,), jnp.float32)
        return 0
    lax.fori_loop(0, ZROWS, _zero_zbuf, 0)
    for k in range(RPT // ZROWS):
        pltpu.sync_copy(zbuf_v, acc_sh.at[pl.ds(s * RPT + k * ZROWS, ZROWS)])
    plsc.subcore_barrier()

    def _chunk(i, _):
        base = wid * EPT + i * CHUNK
        pltpu.sync_copy(src_hbm.at[pl.ds(base, CHUNK)], src_v)
        pltpu.sync_copy(dst_hbm.at[pl.ds(base, CHUNK)], dst_v)
        pltpu.sync_copy(w_hbm.at[pl.ds(base, CHUNK)], w_v)
        pltpu.async_copy(presup_hbm.at[src_v], rows_v, sem).wait()

        def _scale(g, _):
            wv = w_v[pl.ds(g * 16, 16)]
            for e16 in range(16):
                e = g * 16 + e16
                w_s = wv[e16]
                for j in range(LANES):
                    rows_v[e, pl.ds(j * 16, 16)] = rows_v[e, pl.ds(j * 16, 16)] * w_s
            return 0
        lax.fori_loop(0, CHUNK // 16, _scale, 0)

        pltpu.sync_copy(rows_v, acc_sh.at[dst_v], add=True)
        return 0
    lax.fori_loop(0, NCHUNK, _chunk, 0)

    plsc.subcore_barrier()
    pltpu.sync_copy(acc_sh.at[pl.ds(s * RPT, RPT)], out_hbm.at[c, pl.ds(s * RPT, RPT)])


@jax.jit
def kernel(x, edge_index, edge_weight, W):
    pre_sup = pl.pallas_call(
        _matmul_body,
        grid=(10,),
        in_specs=[pl.BlockSpec((N // 10, D), lambda i: (i, 0)),
                  pl.BlockSpec((D, D), lambda i: (0, 0))],
        out_specs=pl.BlockSpec((N // 10, D), lambda i: (i, 0)),
        out_shape=jax.ShapeDtypeStruct((N, D), jnp.float32),
    )(x, W)

    sc_kernel = pl.kernel(
        _sc_body,
        out_type=jax.ShapeDtypeStruct((NC, NPAD, D), jnp.float32),
        mesh=plsc.VectorSubcoreMesh(core_axis_name="c", subcore_axis_name="s"),
        scratch_types=[
            pltpu.VMEM((CHUNK,), jnp.int32),
            pltpu.VMEM((CHUNK,), jnp.int32),
            pltpu.VMEM((CHUNK,), jnp.float32),
            pltpu.VMEM((CHUNK, D), jnp.float32),
            pltpu.VMEM((ZROWS, D), jnp.float32),
            pltpu.VMEM_SHARED((NPAD, D), jnp.float32),
            pltpu.SemaphoreType.DMA,
        ],
    )
    partials = sc_kernel(pre_sup, edge_index[0], edge_index[1], edge_weight)

    out = pl.pallas_call(
        _combine_body,
        grid=(10,),
        in_specs=[pl.BlockSpec((NC, N // 10, D), lambda i: (0, i, 0))],
        out_specs=pl.BlockSpec((N // 10, D), lambda i: (i, 0)),
        out_shape=jax.ShapeDtypeStruct((N, D), jnp.float32),
    )(partials)
    return out


# speedup vs baseline: 6.2170x; 1.3900x over previous
import jax
import jax.numpy as jnp
from jax import lax
from jax.experimental import pallas as pl
from jax.experimental.pallas import tpu as pltpu
from jax.experimental.pallas import tpu_sc as plsc

N = 10000
E = 320000
D = 128
NC = 2
NS = 16
NW = NC * NS
EPT = E // NW
CHUNK = 80
NCHUNK = EPT // CHUNK
NBUF = 4
NPIPE = (NCHUNK // NBUF) * NBUF
NPAD = 10240
RPT = NPAD // NS
LANES = D // 16


def _matmul_body(x_ref, w_ref, o_ref):
    o_ref[...] = jnp.dot(x_ref[...], w_ref[...],
                         preferred_element_type=jnp.float32)


def _combine_body(p_ref, o_ref):
    o_ref[...] = jnp.maximum(p_ref[0] + p_ref[1], 0.0)


def _sc_body(presup_hbm, src_hbm, dst_hbm, w_hbm, out_hbm, *scr):
    src_v = scr[0:NBUF]
    dst_v = scr[NBUF:2 * NBUF]
    w_v = scr[2 * NBUF:3 * NBUF]
    rows_v = scr[3 * NBUF:4 * NBUF]
    gsem = scr[4 * NBUF:5 * NBUF]
    ssem = scr[5 * NBUF:6 * NBUF]
    acc_sh = scr[6 * NBUF]

    c_ax = lax.axis_index("c")
    s_ax = lax.axis_index("s")
    wid = c_ax * NS + s_ax

    def _zero_buf(i, _):
        for j in range(LANES):
            rows_v[0][i, pl.ds(j * 16, 16)] = jnp.zeros((16,), jnp.float32)
        return 0
    lax.fori_loop(0, CHUNK, _zero_buf, 0)
    for k in range(RPT // CHUNK):
        pltpu.sync_copy(rows_v[0], acc_sh.at[pl.ds(s_ax * RPT + k * CHUNK, CHUNK)])
    plsc.subcore_barrier()

    def _copy_idx(b, c):
        base = wid * EPT + c * CHUNK
        pltpu.sync_copy(src_hbm.at[pl.ds(base, CHUNK)], src_v[b])
        pltpu.sync_copy(dst_hbm.at[pl.ds(base, CHUNK)], dst_v[b])
        pltpu.sync_copy(w_hbm.at[pl.ds(base, CHUNK)], w_v[b])

    def _issue_gather(b):
        pltpu.async_copy(presup_hbm.at[src_v[b]], rows_v[b], gsem[b])

    def _wait_gather(b):
        pltpu.make_async_copy(presup_hbm.at[src_v[b]], rows_v[b], gsem[b]).wait()

    def _issue_scatter(b):
        pltpu.async_copy(rows_v[b], acc_sh.at[dst_v[b]], ssem[b], add=True)

    def _wait_scatter(b):
        pltpu.make_async_copy(rows_v[b], acc_sh.at[dst_v[b]], ssem[b]).wait()

    def _scale(b):
        rv = rows_v[b]
        wv_ref = w_v[b]

        def _grp(g, _):
            wv = wv_ref[pl.ds(g * 16, 16)]
            for e16 in range(16):
                e = g * 16 + e16
                w_s = wv[e16]
                for j in range(LANES):
                    rv[e, pl.ds(j * 16, 16)] = rv[e, pl.ds(j * 16, 16)] * w_s
            return 0
        lax.fori_loop(0, CHUNK // 16, _grp, 0)

    for b in range(NBUF):
        _copy_idx(b, b)
        _issue_gather(b)

    def _step(k, _):
        for b in range(NBUF):
            _wait_gather(b)
            _scale(b)
            _issue_scatter(b)

            @pl.when(k < NPIPE // NBUF - 1)
            def _prep():
                _wait_scatter(b)
                _copy_idx(b, k * NBUF + b + NBUF)
                _issue_gather(b)
        return 0
    lax.fori_loop(0, NPIPE // NBUF, _step, 0)

    for b in range(NBUF):
        _wait_scatter(b)
    for c in range(NPIPE, NCHUNK):
        _copy_idx(0, c)
        _issue_gather(0)
        _wait_gather(0)
        _scale(0)
        _issue_scatter(0)
        _wait_scatter(0)

    plsc.subcore_barrier()
    pltpu.sync_copy(acc_sh.at[pl.ds(s_ax * RPT, RPT)],
                    out_hbm.at[c_ax, pl.ds(s_ax * RPT, RPT)])


@jax.jit
def kernel(x, edge_index, edge_weight, W):
    pre_sup = pl.pallas_call(
        _matmul_body,
        grid=(10,),
        in_specs=[pl.BlockSpec((N // 10, D), lambda i: (i, 0)),
                  pl.BlockSpec((D, D), lambda i: (0, 0))],
        out_specs=pl.BlockSpec((N // 10, D), lambda i: (i, 0)),
        out_shape=jax.ShapeDtypeStruct((N, D), jnp.float32),
    )(x, W)

    scratch = (
        [pltpu.VMEM((CHUNK,), jnp.int32)] * NBUF
        + [pltpu.VMEM((CHUNK,), jnp.int32)] * NBUF
        + [pltpu.VMEM((CHUNK,), jnp.float32)] * NBUF
        + [pltpu.VMEM((CHUNK, D), jnp.float32)] * NBUF
        + [pltpu.SemaphoreType.DMA] * NBUF
        + [pltpu.SemaphoreType.DMA] * NBUF
        + [pltpu.VMEM_SHARED((NPAD, D), jnp.float32)]
    )
    sc_kernel = pl.kernel(
        _sc_body,
        out_type=jax.ShapeDtypeStruct((NC, NPAD, D), jnp.float32),
        mesh=plsc.VectorSubcoreMesh(core_axis_name="c", subcore_axis_name="s"),
        scratch_types=scratch,
    )
    partials = sc_kernel(pre_sup, edge_index[0], edge_index[1], edge_weight)

    out = pl.pallas_call(
        _combine_body,
        grid=(10,),
        in_specs=[pl.BlockSpec((NC, N // 10, D), lambda i: (0, i, 0))],
        out_specs=pl.BlockSpec((N // 10, D), lambda i: (i, 0)),
        out_shape=jax.ShapeDtypeStruct((N, D), jnp.float32),
    )(partials)
    return out


# speedup vs baseline: 8.5210x; 1.3706x over previous
import jax
import jax.numpy as jnp
from jax import lax
from jax.experimental import pallas as pl
from jax.experimental.pallas import tpu as pltpu
from jax.experimental.pallas import tpu_sc as plsc

N = 10000
E = 320000
D = 128
NC = 2
NS = 16
NW = NC * NS
EPT = E // NW
PCHUNK = 32
TURNS = 313
EPAD = TURNS * PCHUNK
NBUF = 4
LEAD = 2
NPAD = 10240
RPT = NPAD // NS
LANES = D // 16


def _matmul_body(x_ref, w_ref, o_ref):
    o_ref[...] = jnp.dot(x_ref[...], w_ref[...],
                         preferred_element_type=jnp.float32)


def _combine_body(p_ref, o_ref):
    o_ref[...] = jnp.maximum(p_ref[0] + p_ref[1], 0.0)


def _sc_body(presup_hbm, src_hbm, dst_hbm, w_hbm, out_hbm, *scr):
    rows_v = scr[0:NBUF]
    dbuf_v = scr[NBUF:2 * NBUF]
    gsem = scr[2 * NBUF:3 * NBUF]
    ssem = scr[3 * NBUF:4 * NBUF]
    dsem = scr[4 * NBUF:5 * NBUF]
    src_all, w_all, acc_sh = scr[5 * NBUF:5 * NBUF + 3]

    c_ax = lax.axis_index("c")
    s_ax = lax.axis_index("s")
    wid = c_ax * NS + s_ax

    pltpu.sync_copy(src_hbm.at[pl.ds(wid * EPAD, EPAD)], src_all)
    pltpu.sync_copy(w_hbm.at[pl.ds(wid * EPAD, EPAD)], w_all)

    def _zero_buf(i, _):
        for j in range(LANES):
            rows_v[0][i, pl.ds(j * 16, 16)] = jnp.zeros((16,), jnp.float32)
        return 0
    lax.fori_loop(0, PCHUNK, _zero_buf, 0)
    for k in range(RPT // PCHUNK):
        pltpu.sync_copy(rows_v[0],
                        acc_sh.at[pl.ds(s_ax * RPT + k * PCHUNK, PCHUNK)])
    plsc.subcore_barrier()

    def _issue_gather(b, t):
        pltpu.async_copy(presup_hbm.at[src_all.at[pl.ds(t * PCHUNK, PCHUNK)]],
                         rows_v[b], gsem[b])

    def _wait_gather(b):
        pltpu.make_async_copy(presup_hbm.at[src_all.at[pl.ds(0, PCHUNK)]],
                              rows_v[b], gsem[b]).wait()

    def _issue_scatter(b):
        pltpu.async_copy(rows_v[b], acc_sh.at[dbuf_v[b]], ssem[b], add=True)

    def _wait_scatter(b):
        pltpu.make_async_copy(rows_v[b], acc_sh.at[dbuf_v[b]], ssem[b]).wait()

    def _issue_dst(b, t):
        pltpu.async_copy(dst_hbm.at[pl.ds(wid * EPAD + t * PCHUNK, PCHUNK)],
                         dbuf_v[b], dsem[b])

    def _wait_dst(b):
        pltpu.make_async_copy(dst_hbm.at[pl.ds(0, PCHUNK)], dbuf_v[b],
                              dsem[b]).wait()

    def _scale(b, t):
        rv = rows_v[b]
        for g in range(PCHUNK // 16):
            wv = w_all[pl.ds(t * PCHUNK + g * 16, 16)]
            for e16 in range(16):
                e = g * 16 + e16
                w_s = wv[e16]
                for j in range(LANES):
                    rv[e, pl.ds(j * 16, 16)] = rv[e, pl.ds(j * 16, 16)] * w_s

    for p in range(LEAD):
        _issue_dst(p % NBUF, p)
        _issue_gather(p % NBUF, p)

    def _step(k, _):
        for b in range(NBUF):
            t = k * NBUF + b

            @pl.when(t < TURNS)
            def _proc():
                _wait_gather(b)
                _scale(b, t)
                _wait_dst(b)
                _issue_scatter(b)

            p = t + LEAD
            bp = (b + LEAD) % NBUF

            @pl.when(p < TURNS)
            def _prep():
                @pl.when(p >= NBUF)
                def _drain():
                    _wait_scatter(bp)
                _issue_dst(bp, p)
                _issue_gather(bp, p)
        return 0
    lax.fori_loop(0, (TURNS + NBUF - 1) // NBUF, _step, 0)

    for t in range(TURNS - NBUF, TURNS):
        _wait_scatter(t % NBUF)

    plsc.subcore_barrier()
    pltpu.sync_copy(acc_sh.at[pl.ds(s_ax * RPT, RPT)],
                    out_hbm.at[c_ax, pl.ds(s_ax * RPT, RPT)])


@jax.jit
def kernel(x, edge_index, edge_weight, W):
    pre_sup = pl.pallas_call(
        _matmul_body,
        grid=(10,),
        in_specs=[pl.BlockSpec((N // 10, D), lambda i: (i, 0)),
                  pl.BlockSpec((D, D), lambda i: (0, 0))],
        out_specs=pl.BlockSpec((N // 10, D), lambda i: (i, 0)),
        out_shape=jax.ShapeDtypeStruct((N, D), jnp.float32),
    )(x, W)

    pad = EPAD - EPT
    src2 = jnp.pad(edge_index[0].reshape(NW, EPT), ((0, 0), (0, pad))).reshape(-1)
    dst2 = jnp.pad(edge_index[1].reshape(NW, EPT), ((0, 0), (0, pad)),
                   constant_values=N).reshape(-1)
    w2 = jnp.pad(edge_weight.reshape(NW, EPT), ((0, 0), (0, pad))).reshape(-1)

    scratch = (
        [pltpu.VMEM((PCHUNK, D), jnp.float32)] * NBUF
        + [pltpu.VMEM((PCHUNK,), jnp.int32)] * NBUF
        + [pltpu.SemaphoreType.DMA] * NBUF
        + [pltpu.SemaphoreType.DMA] * NBUF
        + [pltpu.SemaphoreType.DMA] * NBUF
        + [pltpu.VMEM((EPAD,), jnp.int32),
           pltpu.VMEM((EPAD,), jnp.float32),
           pltpu.VMEM_SHARED((NPAD, D), jnp.float32)]
    )
    sc_kernel = pl.kernel(
        _sc_body,
        out_type=jax.ShapeDtypeStruct((NC, NPAD, D), jnp.float32),
        mesh=plsc.VectorSubcoreMesh(core_axis_name="c", subcore_axis_name="s"),
        scratch_types=scratch,
    )
    partials = sc_kernel(pre_sup, src2, dst2, w2)

    out = pl.pallas_call(
        _combine_body,
        grid=(10,),
        in_specs=[pl.BlockSpec((NC, N // 10, D), lambda i: (0, i, 0))],
        out_specs=pl.BlockSpec((N // 10, D), lambda i: (i, 0)),
        out_shape=jax.ShapeDtypeStruct((N, D), jnp.float32),
    )(partials)
    return out


# speedup vs baseline: 8.8339x; 1.0367x over previous
import jax
import jax.numpy as jnp
from jax import lax
from jax.experimental import pallas as pl
from jax.experimental.pallas import tpu as pltpu
from jax.experimental.pallas import tpu_sc as plsc

N = 10000
E = 320000
D = 128
NC = 2
NS = 16
NW = NC * NS
EPT = E // NW
PCHUNK = 48
TURNS = 209
EPAD = TURNS * PCHUNK
NBUF = 4
LEAD = 2
NPAD = 10240
RPT = NPAD // NS
LANES = D // 16


def _matmul_body(x_ref, w_ref, o_ref):
    o_ref[...] = jnp.dot(x_ref[...], w_ref[...],
                         preferred_element_type=jnp.float32)


def _combine_body(p_ref, o_ref):
    o_ref[...] = jnp.maximum(p_ref[0] + p_ref[1], 0.0)


def _sc_body(presup_hbm, src_hbm, dst_hbm, w_hbm, out_hbm, *scr):
    rows_v = scr[0:NBUF]
    dbuf_v = scr[NBUF:2 * NBUF]
    gsem = scr[2 * NBUF:3 * NBUF]
    ssem = scr[3 * NBUF:4 * NBUF]
    dsem = scr[4 * NBUF:5 * NBUF]
    src_all, w_all, acc_sh = scr[5 * NBUF:5 * NBUF + 3]

    c_ax = lax.axis_index("c")
    s_ax = lax.axis_index("s")
    wid = c_ax * NS + s_ax

    pltpu.sync_copy(src_hbm.at[pl.ds(wid * EPAD, EPAD)], src_all)
    pltpu.sync_copy(w_hbm.at[pl.ds(wid * EPAD, EPAD)], w_all)

    def _zero_buf(i, _):
        for j in range(LANES):
            rows_v[0][i, pl.ds(j * 16, 16)] = jnp.zeros((16,), jnp.float32)
        return 0
    lax.fori_loop(0, 32, _zero_buf, 0)
    for k in range(RPT // 32):
        pltpu.sync_copy(rows_v[0].at[pl.ds(0, 32)],
                        acc_sh.at[pl.ds(s_ax * RPT + k * 32, 32)])
    plsc.subcore_barrier()

    def _issue_gather(b, t):
        pltpu.async_copy(presup_hbm.at[src_all.at[pl.ds(t * PCHUNK, PCHUNK)]],
                         rows_v[b], gsem[b])

    def _wait_gather(b):
        pltpu.make_async_copy(presup_hbm.at[src_all.at[pl.ds(0, PCHUNK)]],
                              rows_v[b], gsem[b]).wait()

    def _issue_scatter(b):
        pltpu.async_copy(rows_v[b], acc_sh.at[dbuf_v[b]], ssem[b], add=True)

    def _wait_scatter(b):
        pltpu.make_async_copy(rows_v[b], acc_sh.at[dbuf_v[b]], ssem[b]).wait()

    def _issue_dst(b, t):
        pltpu.async_copy(dst_hbm.at[pl.ds(wid * EPAD + t * PCHUNK, PCHUNK)],
                         dbuf_v[b], dsem[b])

    def _wait_dst(b):
        pltpu.make_async_copy(dst_hbm.at[pl.ds(0, PCHUNK)], dbuf_v[b],
                              dsem[b]).wait()

    def _scale(b, t):
        rv = rows_v[b]

        def _grp(g, _):
            wv = w_all[pl.ds(t * PCHUNK + g * 16, 16)]
            for e16 in range(16):
                w_s = wv[e16]
                for j in range(LANES):
                    idx = (g * 16 + e16, pl.ds(j * 16, 16))
                    rv[idx] = rv[idx] * w_s
            return 0
        lax.fori_loop(0, PCHUNK // 16, _grp, 0)

    for p in range(LEAD):
        _issue_dst(p % NBUF, p)
        _issue_gather(p % NBUF, p)

    def _step(k, _):
        for b in range(NBUF):
            t = k * NBUF + b

            @pl.when(t < TURNS)
            def _proc():
                _wait_gather(b)
                _scale(b, t)
                _wait_dst(b)
                _issue_scatter(b)

            p = t + LEAD
            bp = (b + LEAD) % NBUF

            @pl.when(p < TURNS)
            def _prep():
                @pl.when(p >= NBUF)
                def _drain():
                    _wait_scatter(bp)
                _issue_dst(bp, p)
                _issue_gather(bp, p)
        return 0
    lax.fori_loop(0, (TURNS + NBUF - 1) // NBUF, _step, 0)

    for t in range(TURNS - NBUF, TURNS):
        _wait_scatter(t % NBUF)

    plsc.subcore_barrier()
    pltpu.sync_copy(acc_sh.at[pl.ds(s_ax * RPT, RPT)],
                    out_hbm.at[c_ax, pl.ds(s_ax * RPT, RPT)])


@jax.jit
def kernel(x, edge_index, edge_weight, W):
    pre_sup = pl.pallas_call(
        _matmul_body,
        grid=(10,),
        in_specs=[pl.BlockSpec((N // 10, D), lambda i: (i, 0)),
                  pl.BlockSpec((D, D), lambda i: (0, 0))],
        out_specs=pl.BlockSpec((N // 10, D), lambda i: (i, 0)),
        out_shape=jax.ShapeDtypeStruct((N, D), jnp.float32),
    )(x, W)

    pad = EPAD - EPT
    src2 = jnp.pad(edge_index[0].reshape(NW, EPT), ((0, 0), (0, pad))).reshape(-1)
    dst2 = jnp.pad(edge_index[1].reshape(NW, EPT), ((0, 0), (0, pad)),
                   constant_values=N).reshape(-1)
    w2 = jnp.pad(edge_weight.reshape(NW, EPT), ((0, 0), (0, pad))).reshape(-1)

    scratch = (
        [pltpu.VMEM((PCHUNK, D), jnp.float32)] * NBUF
        + [pltpu.VMEM((PCHUNK,), jnp.int32)] * NBUF
        + [pltpu.SemaphoreType.DMA] * NBUF
        + [pltpu.SemaphoreType.DMA] * NBUF
        + [pltpu.SemaphoreType.DMA] * NBUF
        + [pltpu.VMEM((EPAD,), jnp.int32),
           pltpu.VMEM((EPAD,), jnp.float32),
           pltpu.VMEM_SHARED((NPAD, D), jnp.float32)]
    )
    sc_kernel = pl.kernel(
        _sc_body,
        out_type=jax.ShapeDtypeStruct((NC, NPAD, D), jnp.float32),
        mesh=plsc.VectorSubcoreMesh(core_axis_name="c", subcore_axis_name="s"),
        scratch_types=scratch,
    )
    partials = sc_kernel(pre_sup, src2, dst2, w2)

    out = pl.pallas_call(
        _combine_body,
        grid=(10,),
        in_specs=[pl.BlockSpec((NC, N // 10, D), lambda i: (0, i, 0))],
        out_specs=pl.BlockSpec((N // 10, D), lambda i: (i, 0)),
        out_shape=jax.ShapeDtypeStruct((N, D), jnp.float32),
    )(partials)
    return out
